# Initial kernel scaffold; baseline (speedup 1.0000x reference)
import jax
import jax.numpy as jnp
from jax.experimental import pallas as pl


def kernel(data_x, task_x, edge_index, edge_attr, W_l, b_l, W_r, b_r, W_e, att, W_res, bias, ln_g, ln_b):
    raise NotImplementedError("write your pallas kernel here")


# speedup vs baseline: 40.8332x; 40.8332x over previous
import functools

import jax
import jax.numpy as jnp
from jax import lax
from jax.experimental import pallas as pl
from jax.experimental.pallas import tpu as pltpu
from jax.experimental.pallas import tpu_sc as plsc

N_DATA = 100000
N_TASK = 100000
E = 3200000
HID = 16
NC = 2
NS = 16
NW = NC * NS
EPW = E // NW
C = 800
NWIN = EPW // C
GRP = C // 16
ROWS_PER_TILE = N_TASK // NS
NTP = 100096
DEN_PER_TILE = NTP // NS


def _prep_body(dx_ref, tx_ref, wl_ref, bl_ref, wr_ref, br_ref, xl_ref, xr_ref):
    xl_ref[...] = jnp.dot(dx_ref[...], wl_ref[...],
                          preferred_element_type=jnp.float32) + bl_ref[...]
    xr_ref[...] = jnp.dot(tx_ref[...], wr_ref[...],
                          preferred_element_type=jnp.float32) + br_ref[...]


def _prep(data_x, task_x, W_l, b_l, W_r, b_r):
    B = 2000
    grid = (N_DATA // B,)
    return pl.pallas_call(
        _prep_body,
        grid=grid,
        in_specs=[
            pl.BlockSpec((B, 5), lambda i: (i, 0)),
            pl.BlockSpec((B, 12), lambda i: (i, 0)),
            pl.BlockSpec((5, HID), lambda i: (0, 0)),
            pl.BlockSpec((1, HID), lambda i: (0, 0)),
            pl.BlockSpec((12, HID), lambda i: (0, 0)),
            pl.BlockSpec((1, HID), lambda i: (0, 0)),
        ],
        out_specs=[
            pl.BlockSpec((B, HID), lambda i: (i, 0)),
            pl.BlockSpec((B, HID), lambda i: (i, 0)),
        ],
        out_shape=[
            jax.ShapeDtypeStruct((N_DATA, HID), jnp.float32),
            jax.ShapeDtypeStruct((N_TASK, HID), jnp.float32),
        ],
    )(data_x, task_x, W_l, b_l.reshape(1, HID), W_r, b_r.reshape(1, HID))


_GDN = lax.GatherDimensionNumbers(offset_dims=(), collapsed_slice_dims=(0,),
                                  start_index_map=(0,))


def _shuf(v, idx):
    return lax.gather(v, idx[:, None], _GDN, slice_sizes=(1,),
                      mode=lax.GatherScatterMode.PROMISE_IN_BOUNDS)


def _lane_sum(v):
    for sh in (8, 4, 2, 1):
        v = v + _shuf(v, lax.iota(jnp.int32, 16) ^ sh)
    return v


def _lane_max(v):
    for sh in (8, 4, 2, 1):
        v = jnp.maximum(v, _shuf(v, lax.iota(jnp.int32, 16) ^ sh))
    return v


def _p1_body(xl_hbm, xr_hbm, src_hbm, dst_hbm, ea0_hbm, ea1_hbm, ea2_hbm,
             we_hbm, att_hbm,
             alpha_hbm, tmax_hbm,
             src_v, dst_v, ea0_v, ea1_v, ea2_v, xl_v, xr_v, alpha_v,
             we_v, attw_v, maxs_v, gsem):
    wid = lax.axis_index("s") * NC + lax.axis_index("c")
    estart = wid * EPW
    pltpu.sync_copy(we_hbm, we_v)
    pltpu.sync_copy(att_hbm, attw_v)
    w0 = we_v[0]
    w1 = we_v[1]
    w2 = we_v[2]
    attv = attw_v[0]

    def window(w, maxs):
        base = estart + w * C
        pltpu.sync_copy(src_hbm.at[pl.ds(base, C)], src_v)
        pltpu.sync_copy(dst_hbm.at[pl.ds(base, C)], dst_v)
        pltpu.sync_copy(ea0_hbm.at[pl.ds(base, C)], ea0_v)
        pltpu.sync_copy(ea1_hbm.at[pl.ds(base, C)], ea1_v)
        pltpu.sync_copy(ea2_hbm.at[pl.ds(base, C)], ea2_v)
        cp1 = pltpu.async_copy(xl_hbm.at[src_v], xl_v, gsem)
        cp2 = pltpu.async_copy(xr_hbm.at[dst_v], xr_v, gsem)
        cp1.wait()
        cp2.wait()

        lane = lax.iota(jnp.int32, 16)

        def group(g, ms):
            b16 = g * 16
            ea0g = ea0_v[pl.ds(b16, 16)]
            ea1g = ea1_v[pl.ds(b16, 16)]
            ea2g = ea2_v[pl.ds(b16, 16)]
            av = jnp.zeros((16,), jnp.float32)
            for i in range(16):
                idx = b16 + i
                e = ea0g[i] * w0 + ea1g[i] * w1 + ea2g[i] * w2
                s = xl_v[idx] + xr_v[idx] + e
                t = jnp.maximum(s, 0.2 * s)
                a = _lane_sum(attv * t)
                av = jnp.where(lane == i, a, av)
            alpha_v[pl.ds(b16, 16)] = av
            return jnp.maximum(ms, av)

        maxs = lax.fori_loop(0, GRP, group, maxs)
        pltpu.sync_copy(alpha_v, alpha_hbm.at[pl.ds(base, C)])
        return maxs

    ms = lax.fori_loop(0, NWIN, window,
                       jnp.full((16,), -jnp.inf, jnp.float32))
    maxs_v[...] = ms
    pltpu.sync_copy(maxs_v, tmax_hbm.at[pl.ds(wid * 16, 16)])


def _pass1(xl, xr, src, dst, ea0, ea1, ea2, webb, attb):
    mesh = plsc.VectorSubcoreMesh(core_axis_name="c", subcore_axis_name="s")
    f = functools.partial(
        pl.kernel,
        mesh=mesh,
        out_type=[
            jax.ShapeDtypeStruct((E,), jnp.float32),
            jax.ShapeDtypeStruct((NW * 16,), jnp.float32),
        ],
        compiler_params=pltpu.CompilerParams(use_tc_tiling_on_sc=False),
        scratch_types=[
            pltpu.VMEM((C,), jnp.int32),
            pltpu.VMEM((C,), jnp.int32),
            pltpu.VMEM((C,), jnp.float32),
            pltpu.VMEM((C,), jnp.float32),
            pltpu.VMEM((C,), jnp.float32),
            pltpu.VMEM((C, HID), jnp.float32),
            pltpu.VMEM((C, HID), jnp.float32),
            pltpu.VMEM((C,), jnp.float32),
            pltpu.VMEM((3, 16), jnp.float32),
            pltpu.VMEM((1, 16), jnp.float32),
            pltpu.VMEM((16,), jnp.float32),
            pltpu.SemaphoreType.DMA,
        ],
    )(_p1_body)
    return f(xl, xr, src, dst, ea0, ea1, ea2, webb, attb)


def _p2_body(xl_hbm, src_hbm, dst_hbm, alpha_hbm, tmax_hbm,
             msg_hbm,
             src_v, dst_v, alpha_v, ex_v, xl_v, tm_v, zm_v,
             acc_msg, gsem):
    cid = lax.axis_index("c")
    sid = lax.axis_index("s")
    wid = sid * NC + cid
    estart = wid * EPW

    pltpu.sync_copy(tmax_hbm, tm_v)

    def mred(i, mv):
        return jnp.maximum(mv, tm_v[pl.ds(i * 16, 16)])

    maxv = lax.fori_loop(0, NW, mred, jnp.full((16,), -jnp.inf, jnp.float32))
    gmax = _lane_max(maxv)[0]

    def zfill_m(i, _):
        zm_v[i] = jnp.zeros((16,), jnp.float32)
        return 0

    lax.fori_loop(0, 250, zfill_m, 0)

    def zcopy_m(j, _):
        pltpu.sync_copy(zm_v, acc_msg.at[pl.ds(sid * ROWS_PER_TILE + j * 250, 250)])
        return 0

    lax.fori_loop(0, ROWS_PER_TILE // 250, zcopy_m, 0)

    plsc.subcore_barrier()

    def window(w, _):
        base = estart + w * C
        pltpu.sync_copy(src_hbm.at[pl.ds(base, C)], src_v)
        pltpu.sync_copy(dst_hbm.at[pl.ds(base, C)], dst_v)
        pltpu.sync_copy(alpha_hbm.at[pl.ds(base, C)], alpha_v)
        pltpu.async_copy(xl_hbm.at[src_v], xl_v, gsem).wait()

        def expgrp(g, _):
            b16 = g * 16
            ex_v[pl.ds(b16, 16)] = jnp.exp(alpha_v[pl.ds(b16, 16)] - gmax)
            return 0

        lax.fori_loop(0, GRP, expgrp, 0)

        def scale(g, _):
            b16 = g * 16
            exg = ex_v[pl.ds(b16, 16)]
            for i in range(16):
                idx = b16 + i
                xl_v[idx] = xl_v[idx] * exg[i]
            return 0

        lax.fori_loop(0, GRP, scale, 0)
        pltpu.sync_copy(xl_v, acc_msg.at[dst_v], add=True)
        return 0

    lax.fori_loop(0, NWIN, window, 0)
    plsc.subcore_barrier()

    pltpu.sync_copy(acc_msg.at[pl.ds(sid * ROWS_PER_TILE, ROWS_PER_TILE)],
                    msg_hbm.at[cid, pl.ds(sid * ROWS_PER_TILE, ROWS_PER_TILE)])


def _pass2(xl, src, dst, alpha, tmax):
    mesh = plsc.VectorSubcoreMesh(core_axis_name="c", subcore_axis_name="s")
    f = functools.partial(
        pl.kernel,
        mesh=mesh,
        out_type=[
            jax.ShapeDtypeStruct((NC, N_TASK, HID), jnp.float32),
        ],
        compiler_params=pltpu.CompilerParams(use_tc_tiling_on_sc=False),
        scratch_types=[
            pltpu.VMEM((C,), jnp.int32),
            pltpu.VMEM((C,), jnp.int32),
            pltpu.VMEM((C,), jnp.float32),
            pltpu.VMEM((C,), jnp.float32),
            pltpu.VMEM((C, HID), jnp.float32),
            pltpu.VMEM((NW * 16,), jnp.float32),
            pltpu.VMEM((250, HID), jnp.float32),
            pltpu.VMEM_SHARED((N_TASK, HID), jnp.float32),
            pltpu.SemaphoreType.DMA,
        ],
    )(_p2_body)
    return f(xl, src, dst, alpha, tmax)


def _p2b_body(dst_hbm, alpha_hbm, tmax_hbm,
              den_hbm,
              dst_v, alpha_v, ex_v, tm_v, zd_v, acc_den):
    cid = lax.axis_index("c")
    sid = lax.axis_index("s")
    wid = sid * NC + cid
    estart = wid * EPW

    pltpu.sync_copy(tmax_hbm, tm_v)

    def mred(i, mv):
        return jnp.maximum(mv, tm_v[pl.ds(i * 16, 16)])

    maxv = lax.fori_loop(0, NW, mred, jnp.full((16,), -jnp.inf, jnp.float32))
    gmax = _lane_max(maxv)[0]

    def zfill_d(i, _):
        zd_v[pl.ds(i * 16, 16)] = jnp.zeros((16,), jnp.float32)
        return 0

    lax.fori_loop(0, 64, zfill_d, 0)

    def zcopy_d(j, _):
        pltpu.sync_copy(zd_v, acc_den.at[pl.ds(sid * DEN_PER_TILE + j * 1024, 1024)])
        return 0

    lax.fori_loop(0, 6, zcopy_d, 0)
    pltpu.sync_copy(zd_v.at[pl.ds(0, 112)],
                    acc_den.at[pl.ds(sid * DEN_PER_TILE + 6144, 112)])
    plsc.subcore_barrier()

    def window(w, _):
        base = estart + w * C
        pltpu.sync_copy(dst_hbm.at[pl.ds(base, C)], dst_v)
        pltpu.sync_copy(alpha_hbm.at[pl.ds(base, C)], alpha_v)

        def expgrp(g, _):
            b16 = g * 16
            ex_v[pl.ds(b16, 16)] = jnp.exp(alpha_v[pl.ds(b16, 16)] - gmax)
            return 0

        lax.fori_loop(0, GRP, expgrp, 0)
        pltpu.sync_copy(ex_v, acc_den.at[dst_v], add=True)
        return 0

    lax.fori_loop(0, NWIN, window, 0)
    plsc.subcore_barrier()
    pltpu.sync_copy(acc_den.at[pl.ds(sid * DEN_PER_TILE, DEN_PER_TILE)],
                    den_hbm.at[cid, pl.ds(sid * DEN_PER_TILE, DEN_PER_TILE)])


def _pass2b(dst, alpha, tmax):
    mesh = plsc.VectorSubcoreMesh(core_axis_name="c", subcore_axis_name="s")
    f = functools.partial(
        pl.kernel,
        mesh=mesh,
        out_type=[
            jax.ShapeDtypeStruct((NC, NTP), jnp.float32),
        ],
        compiler_params=pltpu.CompilerParams(use_tc_tiling_on_sc=False),
        scratch_types=[
            pltpu.VMEM((C,), jnp.int32),
            pltpu.VMEM((C,), jnp.float32),
            pltpu.VMEM((C,), jnp.float32),
            pltpu.VMEM((NW * 16,), jnp.float32),
            pltpu.VMEM((1024,), jnp.float32),
            pltpu.VMEM_SHARED((NTP,), jnp.float32),
        ],
    )(_p2b_body)
    return f(dst, alpha, tmax)


def _epi_body(m0_ref, m1_ref, d0_ref, d1_ref, tx_ref, wres_ref, bias_ref,
              lng_ref, lnb_ref, out_ref):
    msg = m0_ref[...] + m1_ref[...]
    den = d0_ref[...] + d1_ref[...]
    out = msg / (den + 1e-30)
    out = out + jnp.dot(tx_ref[...], wres_ref[...],
                        preferred_element_type=jnp.float32) + bias_ref[...]
    mu = jnp.mean(out, axis=-1, keepdims=True)
    var = jnp.mean((out - mu) ** 2, axis=-1, keepdims=True)
    out = (out - mu) * lax.rsqrt(var + 1e-5) * lng_ref[...] + lnb_ref[...]
    out = jnp.maximum(out, 0.01 * out)
    out_ref[...] = jnp.concatenate([out, tx_ref[...]], axis=-1)


def _epilogue(msg_part, den_part, task_x, W_res, bias, ln_g, ln_b):
    B = 2000
    grid = (N_TASK // B,)
    return pl.pallas_call(
        _epi_body,
        grid=grid,
        in_specs=[
            pl.BlockSpec((B, HID), lambda i: (i, 0)),
            pl.BlockSpec((B, HID), lambda i: (i, 0)),
            pl.BlockSpec((B, 1), lambda i: (i, 0)),
            pl.BlockSpec((B, 1), lambda i: (i, 0)),
            pl.BlockSpec((B, 12), lambda i: (i, 0)),
            pl.BlockSpec((12, HID), lambda i: (0, 0)),
            pl.BlockSpec((1, HID), lambda i: (0, 0)),
            pl.BlockSpec((1, HID), lambda i: (0, 0)),
            pl.BlockSpec((1, HID), lambda i: (0, 0)),
        ],
        out_specs=pl.BlockSpec((B, HID + 12), lambda i: (i, 0)),
        out_shape=jax.ShapeDtypeStruct((N_TASK, HID + 12), jnp.float32),
    )(msg_part[0], msg_part[1],
      den_part[0, :N_TASK].reshape(N_TASK, 1),
      den_part[1, :N_TASK].reshape(N_TASK, 1),
      task_x, W_res, bias.reshape(1, HID),
      ln_g.reshape(1, HID), ln_b.reshape(1, HID))


def kernel(data_x, task_x, edge_index, edge_attr,
           W_l, b_l, W_r, b_r, W_e, att, W_res, bias, ln_g, ln_b):
    src = edge_index[0]
    dst = edge_index[1]
    ea0 = edge_attr[:, 0]
    ea1 = edge_attr[:, 1]
    ea2 = edge_attr[:, 2]
    webb = W_e
    attb = att.reshape(1, 16)

    xl, xr = _prep(data_x, task_x, W_l, b_l, W_r, b_r)
    alpha, tmax = _pass1(xl, xr, src, dst, ea0, ea1, ea2, webb, attb)
    (msg_part,) = _pass2(xl, src, dst, alpha, tmax)
    (den_part,) = _pass2b(dst, alpha, tmax)
    return _epilogue(msg_part, den_part, task_x, W_res, bias, ln_g, ln_b)


# speedup vs baseline: 73.8424x; 1.8084x over previous
import functools

import jax
import jax.numpy as jnp
from jax import lax
from jax.experimental import pallas as pl
from jax.experimental.pallas import tpu as pltpu
from jax.experimental.pallas import tpu_sc as plsc

N_DATA = 100000
N_TASK = 100000
E = 3200000
HID = 16
NC = 2
NS = 16
NW = NC * NS
EPW = E // NW
C = 800
NWIN = EPW // C
GRP = C // 16
C2 = 400
NWIN2 = EPW // C2
GRP2 = C2 // 16
ROWS_PER_TILE = N_TASK // NS
NTP = 100096
DEN_PER_TILE = NTP // NS


def _prep_body(dx_ref, tx_ref, wl_ref, bl_ref, wr_ref, br_ref, xl_ref, xr_ref):
    xl_ref[...] = jnp.dot(dx_ref[...], wl_ref[...],
                          preferred_element_type=jnp.float32) + bl_ref[...]
    xr_ref[...] = jnp.dot(tx_ref[...], wr_ref[...],
                          preferred_element_type=jnp.float32) + br_ref[...]


def _prep(data_x, task_x, W_l, b_l, W_r, b_r):
    B = 2000
    grid = (N_DATA // B,)
    return pl.pallas_call(
        _prep_body,
        grid=grid,
        in_specs=[
            pl.BlockSpec((B, 5), lambda i: (i, 0)),
            pl.BlockSpec((B, 12), lambda i: (i, 0)),
            pl.BlockSpec((5, HID), lambda i: (0, 0)),
            pl.BlockSpec((1, HID), lambda i: (0, 0)),
            pl.BlockSpec((12, HID), lambda i: (0, 0)),
            pl.BlockSpec((1, HID), lambda i: (0, 0)),
        ],
        out_specs=[
            pl.BlockSpec((B, HID), lambda i: (i, 0)),
            pl.BlockSpec((B, HID), lambda i: (i, 0)),
        ],
        out_shape=[
            jax.ShapeDtypeStruct((N_DATA, HID), jnp.float32),
            jax.ShapeDtypeStruct((N_TASK, HID), jnp.float32),
        ],
    )(data_x, task_x, W_l, b_l.reshape(1, HID), W_r, b_r.reshape(1, HID))


_GDN = lax.GatherDimensionNumbers(offset_dims=(), collapsed_slice_dims=(0,),
                                  start_index_map=(0,))


def _shuf(v, idx):
    return lax.gather(v, idx[:, None], _GDN, slice_sizes=(1,),
                      mode=lax.GatherScatterMode.PROMISE_IN_BOUNDS)


def _lane_max(v):
    lane = lax.iota(jnp.int32, 16)
    for sh in (8, 4, 2, 1):
        v = jnp.maximum(v, _shuf(v, lane ^ sh))
    return v


def _tree16(qs, ix):
    ix8, ix4, ix2, ix1, m8, m4, m2, m1, brv = ix
    v = [q + _shuf(q, ix8) for q in qs]
    v = [jnp.where(m8, v[2 * i], v[2 * i + 1]) for i in range(8)]
    v = [x + _shuf(x, ix4) for x in v]
    v = [jnp.where(m4, v[2 * i], v[2 * i + 1]) for i in range(4)]
    v = [x + _shuf(x, ix2) for x in v]
    v = [jnp.where(m2, v[2 * i], v[2 * i + 1]) for i in range(2)]
    v = [x + _shuf(x, ix1) for x in v]
    r = jnp.where(m1, v[0], v[1])
    return _shuf(r, brv)


def _mk_ix():
    lane = lax.iota(jnp.int32, 16)
    brv = (((lane & 1) << 3) | ((lane & 2) << 1)
           | ((lane & 4) >> 1) | ((lane & 8) >> 3))
    return (lane ^ 8, lane ^ 4, lane ^ 2, lane ^ 1,
            (lane & 8) == 0, (lane & 4) == 0, (lane & 2) == 0, (lane & 1) == 0,
            brv)


def _p1_body(xl_hbm, xr_hbm, src_hbm, dst_hbm, ea0_hbm, ea1_hbm, ea2_hbm,
             we_hbm, att_hbm,
             alpha_hbm, tmax_hbm,
             src_v0, src_v1, dst_v0, dst_v1,
             ea0_v0, ea0_v1, ea1_v0, ea1_v1, ea2_v0, ea2_v1,
             xl_v0, xl_v1, xr_v0, xr_v1, al_v0, al_v1,
             we_v, attw_v, maxs_v,
             lini0, lini1, line0, line1, gat0, gat1, ao0, ao1):
    wid = lax.axis_index("s") * NC + lax.axis_index("c")
    estart = wid * EPW
    pltpu.sync_copy(we_hbm, we_v)
    pltpu.sync_copy(att_hbm, attw_v)
    w0v = we_v[0]
    w1v = we_v[1]
    w2v = we_v[2]
    attv = attw_v[0]
    ix = _mk_ix()
    spl = [jnp.full((16,), i, jnp.int32) for i in range(16)]

    SRC = (src_v0, src_v1)
    DST = (dst_v0, dst_v1)
    EA0 = (ea0_v0, ea0_v1)
    EA1 = (ea1_v0, ea1_v1)
    EA2 = (ea2_v0, ea2_v1)
    XL = (xl_v0, xl_v1)
    XR = (xr_v0, xr_v1)
    AL = (al_v0, al_v1)
    LINI = (lini0, lini1)
    LINE = (line0, line1)
    GAT = (gat0, gat1)
    AO = (ao0, ao1)

    def lini_start(w, b):
        base = estart + w * C
        pltpu.make_async_copy(src_hbm.at[pl.ds(base, C)], SRC[b], LINI[b]).start()
        pltpu.make_async_copy(dst_hbm.at[pl.ds(base, C)], DST[b], LINI[b]).start()

    def lini_wait(b):
        pltpu.make_async_copy(src_hbm.at[pl.ds(estart, C)], SRC[b], LINI[b]).wait()
        pltpu.make_async_copy(dst_hbm.at[pl.ds(estart, C)], DST[b], LINI[b]).wait()

    def line_start(w, b):
        base = estart + w * C
        pltpu.make_async_copy(ea0_hbm.at[pl.ds(base, C)], EA0[b], LINE[b]).start()
        pltpu.make_async_copy(ea1_hbm.at[pl.ds(base, C)], EA1[b], LINE[b]).start()
        pltpu.make_async_copy(ea2_hbm.at[pl.ds(base, C)], EA2[b], LINE[b]).start()

    def line_wait(b):
        pltpu.make_async_copy(ea0_hbm.at[pl.ds(estart, C)], EA0[b], LINE[b]).wait()
        pltpu.make_async_copy(ea1_hbm.at[pl.ds(estart, C)], EA1[b], LINE[b]).wait()
        pltpu.make_async_copy(ea2_hbm.at[pl.ds(estart, C)], EA2[b], LINE[b]).wait()

    def gat_start(b):
        pltpu.make_async_copy(xl_hbm.at[SRC[b]], XL[b], GAT[b]).start()
        pltpu.make_async_copy(xr_hbm.at[DST[b]], XR[b], GAT[b]).start()

    def gat_wait(b):
        pltpu.make_async_copy(xl_hbm.at[SRC[b]], XL[b], GAT[b]).wait()
        pltpu.make_async_copy(xr_hbm.at[DST[b]], XR[b], GAT[b]).wait()

    def ao_start(w, b):
        base = estart + w * C
        pltpu.make_async_copy(AL[b], alpha_hbm.at[pl.ds(base, C)], AO[b]).start()

    def ao_wait(b):
        pltpu.make_async_copy(AL[b], alpha_hbm.at[pl.ds(estart, C)], AO[b]).wait()

    def compute(b, ms):
        xl_v = XL[b]
        xr_v = XR[b]
        al_v = AL[b]
        ea0_v = EA0[b]
        ea1_v = EA1[b]
        ea2_v = EA2[b]

        def group(g, ms_):
            b16 = g * 16
            ea0g = ea0_v[pl.ds(b16, 16)]
            ea1g = ea1_v[pl.ds(b16, 16)]
            ea2g = ea2_v[pl.ds(b16, 16)]
            qs = []
            for i in range(16):
                e = (_shuf(ea0g, spl[i]) * w0v + _shuf(ea1g, spl[i]) * w1v
                     + _shuf(ea2g, spl[i]) * w2v)
                s = xl_v[b16 + i] + xr_v[b16 + i] + e
                t = jnp.maximum(s, 0.2 * s)
                qs.append(attv * t)
            r = _tree16(qs, ix)
            al_v[pl.ds(b16, 16)] = r
            return jnp.maximum(ms_, r)

        return lax.fori_loop(0, GRP, group, ms)

    def phase(w, p, ms, first):
        nb = 1 - p
        lini_wait(nb)
        gat_start(nb)
        gat_wait(p)

        @pl.when(w + 2 < NWIN)
        def _():
            lini_start(w + 2, p)

        line_wait(p)
        if not first:
            ao_wait(p)
        ms = compute(p, ms)
        ao_start(w, p)

        @pl.when(w + 2 < NWIN)
        def _():
            line_start(w + 2, p)

        return ms

    ninf = jnp.full((16,), -jnp.inf, jnp.float32)
    lini_start(0, 0)
    line_start(0, 0)
    lini_start(1, 1)
    line_start(1, 1)
    lini_wait(0)
    gat_start(0)
    ms = phase(0, 0, ninf, True)
    ms = phase(1, 1, ms, True)

    def dbl(k, ms_):
        w = 2 * k
        ms_ = phase(w, 0, ms_, False)
        ms_ = phase(w + 1, 1, ms_, False)
        return ms_

    ms = lax.fori_loop(1, (NWIN - 1) // 2, dbl, ms)
    gat_wait(0)
    line_wait(0)
    ao_wait(0)
    ms = compute(0, ms)
    ao_start(NWIN - 1, 0)
    ao_wait(1)
    ao_wait(0)
    maxs_v[...] = ms
    pltpu.sync_copy(maxs_v, tmax_hbm.at[pl.ds(wid * 16, 16)])


def _pass1(xl, xr, src, dst, ea0, ea1, ea2, webb, attb):
    mesh = plsc.VectorSubcoreMesh(core_axis_name="c", subcore_axis_name="s")
    f = functools.partial(
        pl.kernel,
        mesh=mesh,
        out_type=[
            jax.ShapeDtypeStruct((E,), jnp.float32),
            jax.ShapeDtypeStruct((NW * 16,), jnp.float32),
        ],
        compiler_params=pltpu.CompilerParams(use_tc_tiling_on_sc=False),
        scratch_types=[
            pltpu.VMEM((C,), jnp.int32),
            pltpu.VMEM((C,), jnp.int32),
            pltpu.VMEM((C,), jnp.int32),
            pltpu.VMEM((C,), jnp.int32),
            pltpu.VMEM((C,), jnp.float32),
            pltpu.VMEM((C,), jnp.float32),
            pltpu.VMEM((C,), jnp.float32),
            pltpu.VMEM((C,), jnp.float32),
            pltpu.VMEM((C,), jnp.float32),
            pltpu.VMEM((C,), jnp.float32),
            pltpu.VMEM((C, HID), jnp.float32),
            pltpu.VMEM((C, HID), jnp.float32),
            pltpu.VMEM((C, HID), jnp.float32),
            pltpu.VMEM((C, HID), jnp.float32),
            pltpu.VMEM((C,), jnp.float32),
            pltpu.VMEM((C,), jnp.float32),
            pltpu.VMEM((3, 16), jnp.float32),
            pltpu.VMEM((1, 16), jnp.float32),
            pltpu.VMEM((16,), jnp.float32),
            pltpu.SemaphoreType.DMA,
            pltpu.SemaphoreType.DMA,
            pltpu.SemaphoreType.DMA,
            pltpu.SemaphoreType.DMA,
            pltpu.SemaphoreType.DMA,
            pltpu.SemaphoreType.DMA,
            pltpu.SemaphoreType.DMA,
            pltpu.SemaphoreType.DMA,
        ],
    )(_p1_body)
    return f(xl, xr, src, dst, ea0, ea1, ea2, webb, attb)


def _p2_body(xl_hbm, src_hbm, dst_hbm, alpha_hbm, tmax_hbm,
             msg_hbm,
             src_v0, src_v1, dst_v0, dst_v1, al_v0, al_v1,
             ex_v, xl_v0, xl_v1, tm_v, zm_v,
             acc_msg, lin0, lin1, gat0, gat1):
    cid = lax.axis_index("c")
    sid = lax.axis_index("s")
    wid = sid * NC + cid
    estart = wid * EPW

    SRC = (src_v0, src_v1)
    DST = (dst_v0, dst_v1)
    AL = (al_v0, al_v1)
    XL = (xl_v0, xl_v1)
    LIN = (lin0, lin1)
    GAT = (gat0, gat1)

    def lin_start(w, b):
        base = estart + w * C2
        pltpu.make_async_copy(src_hbm.at[pl.ds(base, C2)], SRC[b], LIN[b]).start()
        pltpu.make_async_copy(dst_hbm.at[pl.ds(base, C2)], DST[b], LIN[b]).start()
        pltpu.make_async_copy(alpha_hbm.at[pl.ds(base, C2)], AL[b], LIN[b]).start()

    def lin_wait(b):
        pltpu.make_async_copy(src_hbm.at[pl.ds(estart, C2)], SRC[b], LIN[b]).wait()
        pltpu.make_async_copy(dst_hbm.at[pl.ds(estart, C2)], DST[b], LIN[b]).wait()
        pltpu.make_async_copy(alpha_hbm.at[pl.ds(estart, C2)], AL[b], LIN[b]).wait()

    def gat_start(b):
        pltpu.make_async_copy(xl_hbm.at[SRC[b]], XL[b], GAT[b]).start()

    def gat_wait(b):
        pltpu.make_async_copy(xl_hbm.at[SRC[b]], XL[b], GAT[b]).wait()

    lin_start(0, 0)
    lin_start(1, 1)

    pltpu.sync_copy(tmax_hbm, tm_v)

    def mred(i, mv):
        return jnp.maximum(mv, tm_v[pl.ds(i * 16, 16)])

    maxv = lax.fori_loop(0, NW, mred, jnp.full((16,), -jnp.inf, jnp.float32))
    gmax = _lane_max(maxv)[0]

    def zfill_m(i, _):
        zm_v[i] = jnp.zeros((16,), jnp.float32)
        return 0

    lax.fori_loop(0, 250, zfill_m, 0)

    def zcopy_m(j, _):
        pltpu.sync_copy(zm_v, acc_msg.at[pl.ds(sid * ROWS_PER_TILE + j * 250, 250)])
        return 0

    lax.fori_loop(0, ROWS_PER_TILE // 250, zcopy_m, 0)
    plsc.subcore_barrier()

    def compute_scatter(w, p):
        xl_v = XL[p]
        al_v = AL[p]

        def expgrp(g, _):
            b16 = g * 16
            ex_v[pl.ds(b16, 16)] = jnp.exp(al_v[pl.ds(b16, 16)] - gmax)
            return 0

        lax.fori_loop(0, GRP2, expgrp, 0)

        def scale(g, _):
            b16 = g * 16
            exg = ex_v[pl.ds(b16, 16)]
            for i in range(16):
                idx = b16 + i
                xl_v[idx] = xl_v[idx] * exg[i]
            return 0

        lax.fori_loop(0, GRP2, scale, 0)
        pltpu.sync_copy(xl_v, acc_msg.at[DST[p]], add=True)

    def phase(w, p):
        nb = 1 - p

        @pl.when(w + 1 < NWIN2)
        def _():
            lin_wait(nb)
            gat_start(nb)

        gat_wait(p)
        compute_scatter(w, p)

        @pl.when(w + 2 < NWIN2)
        def _():
            lin_start(w + 2, p)

    lin_wait(0)
    gat_start(0)
    phase(0, 0)

    def dbl(k, _):
        w = 2 * k + 1
        phase(w, 1)
        phase(w + 1, 0)
        return 0

    lax.fori_loop(0, (NWIN2 - 1) // 2, dbl, 0)
    if (NWIN2 - 1) % 2 == 1:
        phase(NWIN2 - 1, 1)
    plsc.subcore_barrier()

    pltpu.sync_copy(acc_msg.at[pl.ds(sid * ROWS_PER_TILE, ROWS_PER_TILE)],
                    msg_hbm.at[cid, pl.ds(sid * ROWS_PER_TILE, ROWS_PER_TILE)])


def _pass2(xl, src, dst, alpha, tmax):
    mesh = plsc.VectorSubcoreMesh(core_axis_name="c", subcore_axis_name="s")
    f = functools.partial(
        pl.kernel,
        mesh=mesh,
        out_type=[
            jax.ShapeDtypeStruct((NC, N_TASK, HID), jnp.float32),
        ],
        compiler_params=pltpu.CompilerParams(use_tc_tiling_on_sc=False),
        scratch_types=[
            pltpu.VMEM((C2,), jnp.int32),
            pltpu.VMEM((C2,), jnp.int32),
            pltpu.VMEM((C2,), jnp.int32),
            pltpu.VMEM((C2,), jnp.int32),
            pltpu.VMEM((C2,), jnp.float32),
            pltpu.VMEM((C2,), jnp.float32),
            pltpu.VMEM((C2,), jnp.float32),
            pltpu.VMEM((C2, HID), jnp.float32),
            pltpu.VMEM((C2, HID), jnp.float32),
            pltpu.VMEM((NW * 16,), jnp.float32),
            pltpu.VMEM((250, HID), jnp.float32),
            pltpu.VMEM_SHARED((N_TASK, HID), jnp.float32),
            pltpu.SemaphoreType.DMA,
            pltpu.SemaphoreType.DMA,
            pltpu.SemaphoreType.DMA,
            pltpu.SemaphoreType.DMA,
        ],
    )(_p2_body)
    return f(xl, src, dst, alpha, tmax)


def _p2b_body(dst_hbm, alpha_hbm, tmax_hbm,
              den_hbm,
              dst_v0, dst_v1, al_v0, al_v1, ex_v, tm_v, zd_v,
              acc_den, lin0, lin1):
    cid = lax.axis_index("c")
    sid = lax.axis_index("s")
    wid = sid * NC + cid
    estart = wid * EPW

    DST = (dst_v0, dst_v1)
    AL = (al_v0, al_v1)
    LIN = (lin0, lin1)

    def lin_start(w, b):
        base = estart + w * C
        pltpu.make_async_copy(dst_hbm.at[pl.ds(base, C)], DST[b], LIN[b]).start()
        pltpu.make_async_copy(alpha_hbm.at[pl.ds(base, C)], AL[b], LIN[b]).start()

    def lin_wait(b):
        pltpu.make_async_copy(dst_hbm.at[pl.ds(estart, C)], DST[b], LIN[b]).wait()
        pltpu.make_async_copy(alpha_hbm.at[pl.ds(estart, C)], AL[b], LIN[b]).wait()

    lin_start(0, 0)
    lin_start(1, 1)

    pltpu.sync_copy(tmax_hbm, tm_v)

    def mred(i, mv):
        return jnp.maximum(mv, tm_v[pl.ds(i * 16, 16)])

    maxv = lax.fori_loop(0, NW, mred, jnp.full((16,), -jnp.inf, jnp.float32))
    gmax = _lane_max(maxv)[0]

    def zfill_d(i, _):
        zd_v[pl.ds(i * 16, 16)] = jnp.zeros((16,), jnp.float32)
        return 0

    lax.fori_loop(0, 64, zfill_d, 0)

    def zcopy_d(j, _):
        pltpu.sync_copy(zd_v, acc_den.at[pl.ds(sid * DEN_PER_TILE + j * 1024, 1024)])
        return 0

    lax.fori_loop(0, 6, zcopy_d, 0)
    pltpu.sync_copy(zd_v.at[pl.ds(0, 112)],
                    acc_den.at[pl.ds(sid * DEN_PER_TILE + 6144, 112)])
    plsc.subcore_barrier()

    def phase(w, p):
        lin_wait(p)
        al_v = AL[p]

        def expgrp(g, _):
            b16 = g * 16
            ex_v[pl.ds(b16, 16)] = jnp.exp(al_v[pl.ds(b16, 16)] - gmax)
            return 0

        lax.fori_loop(0, GRP, expgrp, 0)
        pltpu.sync_copy(ex_v, acc_den.at[DST[p]], add=True)

        @pl.when(w + 2 < NWIN)
        def _():
            lin_start(w + 2, p)

    phase(0, 0)

    def dbl(k, _):
        w = 2 * k + 1
        phase(w, 1)
        phase(w + 1, 0)
        return 0

    lax.fori_loop(0, (NWIN - 1) // 2, dbl, 0)
    plsc.subcore_barrier()
    pltpu.sync_copy(acc_den.at[pl.ds(sid * DEN_PER_TILE, DEN_PER_TILE)],
                    den_hbm.at[cid, pl.ds(sid * DEN_PER_TILE, DEN_PER_TILE)])


def _pass2b(dst, alpha, tmax):
    mesh = plsc.VectorSubcoreMesh(core_axis_name="c", subcore_axis_name="s")
    f = functools.partial(
        pl.kernel,
        mesh=mesh,
        out_type=[
            jax.ShapeDtypeStruct((NC, NTP), jnp.float32),
        ],
        compiler_params=pltpu.CompilerParams(use_tc_tiling_on_sc=False),
        scratch_types=[
            pltpu.VMEM((C,), jnp.int32),
            pltpu.VMEM((C,), jnp.int32),
            pltpu.VMEM((C,), jnp.float32),
            pltpu.VMEM((C,), jnp.float32),
            pltpu.VMEM((C,), jnp.float32),
            pltpu.VMEM((NW * 16,), jnp.float32),
            pltpu.VMEM((1024,), jnp.float32),
            pltpu.VMEM_SHARED((NTP,), jnp.float32),
            pltpu.SemaphoreType.DMA,
            pltpu.SemaphoreType.DMA,
        ],
    )(_p2b_body)
    return f(dst, alpha, tmax)


def _epi_body(m0_ref, m1_ref, d0_ref, d1_ref, tx_ref, wres_ref, bias_ref,
              lng_ref, lnb_ref, out_ref):
    msg = m0_ref[...] + m1_ref[...]
    den = d0_ref[...] + d1_ref[...]
    out = msg / (den + 1e-30)
    out = out + jnp.dot(tx_ref[...], wres_ref[...],
                        preferred_element_type=jnp.float32) + bias_ref[...]
    mu = jnp.mean(out, axis=-1, keepdims=True)
    var = jnp.mean((out - mu) ** 2, axis=-1, keepdims=True)
    out = (out - mu) * lax.rsqrt(var + 1e-5) * lng_ref[...] + lnb_ref[...]
    out = jnp.maximum(out, 0.01 * out)
    out_ref[...] = jnp.concatenate([out, tx_ref[...]], axis=-1)


def _epilogue(msg_part, den_part, task_x, W_res, bias, ln_g, ln_b):
    B = 2000
    grid = (N_TASK // B,)
    return pl.pallas_call(
        _epi_body,
        grid=grid,
        in_specs=[
            pl.BlockSpec((B, HID), lambda i: (i, 0)),
            pl.BlockSpec((B, HID), lambda i: (i, 0)),
            pl.BlockSpec((B, 1), lambda i: (i, 0)),
            pl.BlockSpec((B, 1), lambda i: (i, 0)),
            pl.BlockSpec((B, 12), lambda i: (i, 0)),
            pl.BlockSpec((12, HID), lambda i: (0, 0)),
            pl.BlockSpec((1, HID), lambda i: (0, 0)),
            pl.BlockSpec((1, HID), lambda i: (0, 0)),
            pl.BlockSpec((1, HID), lambda i: (0, 0)),
        ],
        out_specs=pl.BlockSpec((B, HID + 12), lambda i: (i, 0)),
        out_shape=jax.ShapeDtypeStruct((N_TASK, HID + 12), jnp.float32),
    )(msg_part[0], msg_part[1],
      den_part[0, :N_TASK].reshape(N_TASK, 1),
      den_part[1, :N_TASK].reshape(N_TASK, 1),
      task_x, W_res, bias.reshape(1, HID),
      ln_g.reshape(1, HID), ln_b.reshape(1, HID))


def kernel(data_x, task_x, edge_index, edge_attr,
           W_l, b_l, W_r, b_r, W_e, att, W_res, bias, ln_g, ln_b):
    src = edge_index[0]
    dst = edge_index[1]
    ea0 = edge_attr[:, 0]
    ea1 = edge_attr[:, 1]
    ea2 = edge_attr[:, 2]
    webb = W_e
    attb = att.reshape(1, 16)

    xl, xr = _prep(data_x, task_x, W_l, b_l, W_r, b_r)
    alpha, tmax = _pass1(xl, xr, src, dst, ea0, ea1, ea2, webb, attb)
    (msg_part,) = _pass2(xl, src, dst, alpha, tmax)
    (den_part,) = _pass2b(dst, alpha, tmax)
    return _epilogue(msg_part, den_part, task_x, W_res, bias, ln_g, ln_b)
